# baseline (speedup 1.0000x reference)
import functools

import jax
import jax.numpy as jnp
from jax import lax
from jax.experimental import pallas as pl
from jax.experimental.pallas import tpu as pltpu
from jax.experimental.pallas import tpu_sc as plsc

B = 1024
L = 200
E = 128
NC = 2
NS = 16
NW = NC * NS
ROWS = B * L
RPW = ROWS // NW
CHUNK = 40
NCH = RPW // CHUNK
NBUF = 10
PE_ROWS = L + CHUNK


LEAD = 5


def _body(seq_hbm, pe_hbm, table_hbm, out_hbm, *scratch):
    bufs = scratch[:NBUF]
    gsems = scratch[NBUF:2 * NBUF]
    osems = scratch[2 * NBUF:3 * NBUF]
    idx_v = scratch[3 * NBUF]
    pe_v = scratch[3 * NBUF + 1]
    wid = lax.axis_index("s") * NC + lax.axis_index("c")
    base = wid * RPW

    pltpu.sync_copy(seq_hbm.at[wid], idx_v)
    pltpu.sync_copy(pe_hbm, pe_v)

    def out_slice(c):
        return out_hbm.at[pl.ds(base + c * CHUNK, CHUNK)]

    for b in range(LEAD):
        pltpu.async_copy(table_hbm.at[idx_v.at[b]], bufs[b], gsems[b])

    def add_pe(buf, poff):
        @plsc.parallel_loop(0, CHUNK, 1, unroll=8)
        def row(i):
            for j in range(E // 16):
                sl = pl.ds(j * 16, 16)
                plsc.addupdate(buf.at[i, sl], pe_v[poff + i, sl])

    def group(g):
        for b in range(NBUF):
            c = g + b
            bn = (b + LEAD) % NBUF
            cn = c + LEAD

            @pl.when(cn < NCH)
            def _():
                @pl.when(cn >= NBUF)
                def _():
                    pltpu.make_async_copy(
                        bufs[bn], out_slice(cn - NBUF), osems[bn]).wait()
                pltpu.async_copy(
                    table_hbm.at[idx_v.at[cn]], bufs[bn], gsems[bn])

            pltpu.make_async_copy(
                table_hbm.at[idx_v.at[c]], bufs[b], gsems[b]).wait()
            add_pe(bufs[b], lax.rem(c * CHUNK, L))
            pltpu.async_copy(bufs[b], out_slice(c), osems[b])

    pl.loop(0, NCH, step=NBUF, unroll=1)(group)

    for b in range(NBUF):
        cd = NCH - NBUF + b
        pltpu.make_async_copy(bufs[b], out_slice(cd), osems[b]).wait()


@functools.partial(jax.jit, static_argnames=())
def _run(seq, table, pe_slice):
    grid_kernel = pl.kernel(
        _body,
        mesh=plsc.VectorSubcoreMesh(core_axis_name="c", subcore_axis_name="s",
                                    num_cores=NC, num_subcores=NS),
        out_type=jax.ShapeDtypeStruct((ROWS, E), jnp.float32),
        scratch_types=(
            [pltpu.VMEM((CHUNK, E), jnp.float32) for _ in range(NBUF)]
            + [pltpu.SemaphoreType.DMA for _ in range(2 * NBUF)]
            + [pltpu.VMEM((NCH, CHUNK), jnp.int32),
               pltpu.VMEM((PE_ROWS, E), jnp.float32)]
        ),
    )
    return grid_kernel(seq, pe_slice, table)


def kernel(sequence, table, pe):
    seq = sequence.astype(jnp.int32).reshape(NW, NCH, CHUNK)
    pe_slice = pe.reshape(-1, E)[:L]
    pe_ext = jnp.concatenate([pe_slice, pe_slice[:CHUNK]], axis=0)
    out = _run(seq, table, pe_ext)
    return out.reshape(B, L, E)


# speedup vs baseline: 1.1644x; 1.1644x over previous
import functools

import jax
import jax.numpy as jnp
from jax import lax
from jax.experimental import pallas as pl
from jax.experimental.pallas import tpu as pltpu
from jax.experimental.pallas import tpu_sc as plsc

B = 1024
L = 200
E = 128
NC = 2
NS = 16
NW = NC * NS
ROWS = B * L
RPW = ROWS // NW
CHUNK = 40
NCH = RPW // CHUNK
NBUF = 10
PE_ROWS = L + CHUNK


LEAD = 5


def _body(seq_hbm, pe_hbm, table_hbm, out_hbm, *scratch):
    bufs = scratch[:NBUF]
    gsems = scratch[NBUF:2 * NBUF]
    osems = scratch[2 * NBUF:3 * NBUF]
    idx_v = scratch[3 * NBUF]
    pe_v = scratch[3 * NBUF + 1]
    wid = lax.axis_index("s") * NC + lax.axis_index("c")
    base = wid * RPW

    pltpu.sync_copy(seq_hbm.at[wid], idx_v)
    pltpu.sync_copy(pe_hbm, pe_v)

    def out_slice(c):
        return out_hbm.at[pl.ds(base + c * CHUNK, CHUNK)]

    for b in range(LEAD):
        pltpu.async_copy(table_hbm.at[idx_v.at[b]], bufs[b], gsems[b])

    def add_pe(buf, poff):
        @plsc.parallel_loop(0, CHUNK, 1, unroll=4)
        def row(i):
            for j in range(E // 16):
                sl = pl.ds(j * 16, 16)
                plsc.addupdate(buf.at[i, sl], pe_v[poff + i, sl])

    def group(g):
        for b in range(NBUF):
            c = g + b
            bn = (b + LEAD) % NBUF
            cn = c + LEAD

            @pl.when(cn < NCH)
            def _():
                @pl.when(cn >= NBUF)
                def _():
                    pltpu.make_async_copy(
                        bufs[bn], out_slice(cn - NBUF), osems[bn]).wait()
                pltpu.async_copy(
                    table_hbm.at[idx_v.at[cn]], bufs[bn], gsems[bn])

            pltpu.make_async_copy(
                table_hbm.at[idx_v.at[c]], bufs[b], gsems[b]).wait()
            add_pe(bufs[b], lax.rem(c * CHUNK, L))
            pltpu.async_copy(bufs[b], out_slice(c), osems[b])

    pl.loop(0, NCH, step=NBUF, unroll=1)(group)

    for b in range(NBUF):
        cd = NCH - NBUF + b
        pltpu.make_async_copy(bufs[b], out_slice(cd), osems[b]).wait()


@functools.partial(jax.jit, static_argnames=())
def _run(seq, table, pe_slice):
    grid_kernel = pl.kernel(
        _body,
        mesh=plsc.VectorSubcoreMesh(core_axis_name="c", subcore_axis_name="s",
                                    num_cores=NC, num_subcores=NS),
        out_type=jax.ShapeDtypeStruct((ROWS, E), jnp.float32),
        scratch_types=(
            [pltpu.VMEM((CHUNK, E), jnp.float32) for _ in range(NBUF)]
            + [pltpu.SemaphoreType.DMA for _ in range(2 * NBUF)]
            + [pltpu.VMEM((NCH, CHUNK), jnp.int32),
               pltpu.VMEM((PE_ROWS, E), jnp.float32)]
        ),
    )
    return grid_kernel(seq, pe_slice, table)


def kernel(sequence, table, pe):
    seq = sequence.astype(jnp.int32).reshape(NW, NCH, CHUNK)
    pe_slice = pe.reshape(-1, E)[:L]
    pe_ext = jnp.concatenate([pe_slice, pe_slice[:CHUNK]], axis=0)
    out = _run(seq, table, pe_ext)
    return out.reshape(B, L, E)


# speedup vs baseline: 1.1934x; 1.0250x over previous
import functools

import jax
import jax.numpy as jnp
from jax import lax
from jax.experimental import pallas as pl
from jax.experimental.pallas import tpu as pltpu
from jax.experimental.pallas import tpu_sc as plsc

B = 1024
L = 200
E = 128
NC = 2
NS = 16
NW = NC * NS
ROWS = B * L
RPW = ROWS // NW
CHUNK = 40
NCH = RPW // CHUNK
NBUF = 10
PE_ROWS = L


LEAD = 5


def _body(seq_hbm, pe_hbm, table_hbm, out_hbm, *scratch):
    bufs = scratch[:NBUF]
    gsems = scratch[NBUF:2 * NBUF]
    osems = scratch[2 * NBUF:3 * NBUF]
    idx_v = scratch[3 * NBUF]
    pe_v = scratch[3 * NBUF + 1]
    wid = lax.axis_index("s") * NC + lax.axis_index("c")
    base = wid * RPW

    pltpu.sync_copy(seq_hbm.at[wid], idx_v)
    pltpu.sync_copy(pe_hbm, pe_v)

    def out_slice(c):
        return out_hbm.at[pl.ds(base + c * CHUNK, CHUNK)]

    for b in range(LEAD):
        pltpu.async_copy(table_hbm.at[idx_v.at[b]], bufs[b], gsems[b])

    def add_pe(buf, poff):
        @plsc.parallel_loop(0, CHUNK, 1, unroll=4)
        def row(i):
            for j in range(E // 16):
                sl = pl.ds(j * 16, 16)
                plsc.addupdate(buf.at[i, sl], pe_v[poff + i, sl])

    def group(g):
        for b in range(NBUF):
            c = g + b
            bn = (b + LEAD) % NBUF
            cn = c + LEAD

            @pl.when(cn < NCH)
            def _():
                @pl.when(cn >= NBUF)
                def _():
                    pltpu.make_async_copy(
                        bufs[bn], out_slice(cn - NBUF), osems[bn]).wait()
                pltpu.async_copy(
                    table_hbm.at[idx_v.at[cn]], bufs[bn], gsems[bn])

            pltpu.make_async_copy(
                table_hbm.at[idx_v.at[c]], bufs[b], gsems[b]).wait()
            add_pe(bufs[b], lax.rem(c, L // CHUNK) * CHUNK)
            pltpu.async_copy(bufs[b], out_slice(c), osems[b])

    pl.loop(0, NCH, step=NBUF, unroll=1)(group)

    for b in range(NBUF):
        cd = NCH - NBUF + b
        pltpu.make_async_copy(bufs[b], out_slice(cd), osems[b]).wait()


@functools.partial(jax.jit, static_argnames=())
def _run(seq, table, pe_slice):
    grid_kernel = pl.kernel(
        _body,
        mesh=plsc.VectorSubcoreMesh(core_axis_name="c", subcore_axis_name="s",
                                    num_cores=NC, num_subcores=NS),
        out_type=jax.ShapeDtypeStruct((ROWS, E), jnp.float32),
        scratch_types=(
            [pltpu.VMEM((CHUNK, E), jnp.float32) for _ in range(NBUF)]
            + [pltpu.SemaphoreType.DMA for _ in range(2 * NBUF)]
            + [pltpu.VMEM((NCH, CHUNK), jnp.int32),
               pltpu.VMEM((PE_ROWS, E), jnp.float32)]
        ),
    )
    return grid_kernel(seq, pe_slice, table)


def kernel(sequence, table, pe):
    seq = sequence.astype(jnp.int32).reshape(NW, NCH, CHUNK)
    pe_slice = pe.reshape(-1, E)[:L]
    out = _run(seq, table, pe_slice)
    return out.reshape(B, L, E)
